# baseline (speedup 1.0000x reference)
import functools

import jax
import jax.numpy as jnp
from jax import lax
from jax.experimental import pallas as pl
from jax.experimental.pallas import tpu as pltpu
from jax.experimental.pallas import tpu_sc as plsc

N = 10000
E = 160000
H = 256
HH = 128
EPS = 1e-5

NS = 16
EPT = E // NS
CH = 80
EPADT = 240
EPT2 = EPT + EPADT
NCH = EPT2 // CH
G = 16
NGRP = NCH // G
ACC_N = N + 8
DCH = 125
DNCH = (E // (2 * NS)) // DCH
ROWS_A = 624

_MESH = plsc.VectorSubcoreMesh(core_axis_name="c", subcore_axis_name="s")


def _sc_aggregate(h_lo, h_hi, src_r, dst_r, zeros_lo):
    out_t = jax.ShapeDtypeStruct((N, HH), jnp.float32)

    @functools.partial(
        pl.kernel,
        out_type=(out_t, out_t),
        mesh=_MESH,
        scratch_types=[
            pltpu.VMEM_SHARED((ACC_N, HH), jnp.float32),
            pltpu.VMEM((G, CH), jnp.int32),
            pltpu.VMEM((G, CH), jnp.int32),
            pltpu.VMEM((G, CH), jnp.int32),
            pltpu.VMEM((G, CH), jnp.int32),
            pltpu.VMEM((CH, HH), jnp.float32),
            pltpu.VMEM((CH, HH), jnp.float32),
            pltpu.VMEM((CH, HH), jnp.float32),
            pltpu.VMEM((CH, HH), jnp.float32),
            pltpu.SemaphoreType.DMA,
            pltpu.SemaphoreType.DMA,
            pltpu.SemaphoreType.DMA,
            pltpu.SemaphoreType.DMA,
            pltpu.SemaphoreType.DMA,
            pltpu.SemaphoreType.DMA,
            pltpu.SemaphoreType.DMA,
            pltpu.SemaphoreType.DMA,
            pltpu.SemaphoreType.DMA,
            pltpu.SemaphoreType.DMA,
        ],
    )
    def k(hlo, hhi, srcr, dstr, zlo, olo, ohi, acc, si0, si1, di0, di1,
          r0, r1, r2, r3, gs0, gs1, gs2, gs3, ss0, ss1, ss2, ss3,
          isem_s, isem_d):
        c = lax.axis_index("c")
        s = lax.axis_index("s")
        base = s * ROWS_A
        pltpu.sync_copy(zlo.at[pl.ds(base, ROWS_A)], acc.at[pl.ds(base, ROWS_A)])

        @pl.when(s == NS - 1)
        def _():
            pltpu.sync_copy(zlo.at[pl.ds(N - 16, 16)], acc.at[pl.ds(N - 16, 16)])

        pltpu.sync_copy(srcr.at[s, 0], si0)
        pltpu.sync_copy(dstr.at[s, 0], di0)
        plsc.subcore_barrier()

        def edge_loop(h_src):
            rows = (r0, r1, r2, r3)
            gs = (gs0, gs1, gs2, gs3)
            ss = (ss0, ss1, ss2, ss3)

            def g_start(ib, l, b):
                pltpu.make_async_copy(h_src.at[ib.at[l]], rows[b], gs[b]).start()

            def g_wait(ib, l, b):
                pltpu.make_async_copy(h_src.at[ib.at[l]], rows[b], gs[b]).wait()

            def s_start(ib, l, b):
                pltpu.make_async_copy(rows[b], acc.at[ib.at[l]], ss[b]).start(add=True)

            def s_wait(b):
                pltpu.make_async_copy(rows[b], acc.at[di0.at[0]], ss[b]).wait()

            g_start(si0, 0, 0)
            g_start(si0, 1, 1)

            @pl.loop(0, NGRP // 2)
            def _(gg):
                for half in range(2):
                    sib, dib = (si0, di0) if half == 0 else (si1, di1)
                    nsib, ndib = (si1, di1) if half == 0 else (si0, di0)
                    grp = 2 * gg + half
                    have_next = grp + 1 < NGRP

                    def chunk(l, b, swait_fn):
                        g_wait(sib, l, b)
                        swait_fn((b + 2) % 4)
                        g_start(sib, l + 2, (b + 2) % 4)
                        s_start(dib, l, b)

                    if half == 0:
                        def swait0(b):
                            @pl.when(gg > 0)
                            def _():
                                s_wait(b)
                    else:
                        swait0 = s_wait
                    chunk(0, 0, swait0)
                    chunk(1, 1, swait0)
                    @pl.when(have_next)
                    def _():
                        pltpu.make_async_copy(srcr.at[s, grp + 1], nsib,
                                              isem_s).start()
                        pltpu.make_async_copy(dstr.at[s, grp + 1], ndib,
                                              isem_d).start()

                    @pl.loop(0, (G - 4) // 4)
                    def _(q):
                        for kk in range(4):
                            chunk(2 + 4 * q + kk, (2 + kk) % 4, s_wait)

                    g_wait(sib, G - 2, 2)
                    s_wait(0)

                    @pl.when(have_next)
                    def _():
                        pltpu.make_async_copy(srcr.at[s, grp + 1], nsib,
                                              isem_s).wait()
                        pltpu.make_async_copy(dstr.at[s, grp + 1], ndib,
                                              isem_d).wait()
                        g_start(nsib, 0, 0)

                    s_start(dib, G - 2, 2)
                    g_wait(sib, G - 1, 3)
                    s_wait(1)

                    @pl.when(have_next)
                    def _():
                        g_start(nsib, 1, 1)

                    s_start(dib, G - 1, 3)

            s_wait(2)
            s_wait(3)

        @pl.when(c == 0)
        def _():
            edge_loop(hlo)

        @pl.when(c == 1)
        def _():
            edge_loop(hhi)

        plsc.subcore_barrier()

        def writeout(o):
            pltpu.sync_copy(acc.at[pl.ds(base, ROWS_A)], o.at[pl.ds(base, ROWS_A)])

            @pl.when(s == NS - 1)
            def _():
                pltpu.sync_copy(acc.at[pl.ds(N - 16, 16)], o.at[pl.ds(N - 16, 16)])

        @pl.when(c == 0)
        def _():
            writeout(olo)

        @pl.when(c == 1)
        def _():
            writeout(ohi)

    return k(h_lo, h_hi, src_r, dst_r, zeros_lo)


def _sc_degree(dst_r2, zeros_lo, ones40):
    out_t = jax.ShapeDtypeStruct((N, HH), jnp.float32)

    @functools.partial(
        pl.kernel,
        out_type=(out_t, out_t),
        mesh=_MESH,
        scratch_types=[
            pltpu.VMEM_SHARED((N, HH), jnp.float32),
            pltpu.VMEM((DNCH, DCH), jnp.int32),
            pltpu.VMEM((DCH, HH), jnp.float32),
            pltpu.SemaphoreType.DMA,
        ],
    )
    def k(dstr, zlo, ones_hbm, o0, o1, acc, didx, ones, ssem):
        c = lax.axis_index("c")
        s = lax.axis_index("s")
        base = s * ROWS_A
        pltpu.sync_copy(zlo.at[pl.ds(base, ROWS_A)], acc.at[pl.ds(base, ROWS_A)])

        @pl.when(s == NS - 1)
        def _():
            pltpu.sync_copy(zlo.at[pl.ds(N - 16, 16)], acc.at[pl.ds(N - 16, 16)])

        tid = c * NS + s
        pltpu.sync_copy(dstr.at[tid], didx)
        pltpu.sync_copy(ones_hbm, ones)
        plsc.subcore_barrier()

        @pl.loop(0, DNCH)
        def _(j):
            pltpu.make_async_copy(ones, acc.at[didx.at[j]], ssem).start(add=True)

        @pl.loop(0, DNCH)
        def _(j):
            pltpu.make_async_copy(ones, acc.at[didx.at[0]], ssem).wait()

        plsc.subcore_barrier()

        def writeout(o):
            pltpu.sync_copy(acc.at[pl.ds(base, ROWS_A)], o.at[pl.ds(base, ROWS_A)])

            @pl.when(s == NS - 1)
            def _():
                pltpu.sync_copy(acc.at[pl.ds(N - 16, 16)], o.at[pl.ds(N - 16, 16)])

        @pl.when(c == 0)
        def _():
            writeout(o0)

        @pl.when(c == 1)
        def _():
            writeout(o1)

    return k(dst_r2, zeros_lo, ones40)


RB = 1000
NRB = N // RB


def _row_spec():
    return pl.BlockSpec((RB, HH), lambda i: (i, 0))


def _tc_layer_a(agg_lo, agg_hi, h_lo, h_hi, deg, Wl, bl, Wr):
    def body(al, ah, hl, hh, dg, wl, b, wr, out_ref, st_ref, accs):
        i = pl.program_id(0)
        rdeg = 1.0 / jnp.maximum(dg[...], 1.0)
        ml = al[...] * rdeg
        mh = ah[...] * rdeg
        out = (
            jnp.dot(ml, wl[0:HH, :], preferred_element_type=jnp.float32)
            + jnp.dot(mh, wl[HH:H, :], preferred_element_type=jnp.float32)
            + jnp.dot(hl[...], wr[0:HH, :], preferred_element_type=jnp.float32)
            + jnp.dot(hh[...], wr[HH:H, :], preferred_element_type=jnp.float32)
            + b[...]
        )
        out_ref[...] = out

        @pl.when(i == 0)
        def _():
            accs[...] = jnp.zeros_like(accs)

        accs[0:1, :] += jnp.sum(out, axis=0, keepdims=True)
        accs[1:2, :] += jnp.sum(out * out, axis=0, keepdims=True)

        @pl.when(i == NRB - 1)
        def _():
            st_ref[...] = accs[...]

    return pl.pallas_call(
        body,
        grid=(NRB,),
        in_specs=[
            _row_spec(), _row_spec(), _row_spec(), _row_spec(),
            pl.BlockSpec((RB, 1), lambda i: (i, 0)),
            pl.BlockSpec((H, H), lambda i: (0, 0)),
            pl.BlockSpec((1, H), lambda i: (0, 0)),
            pl.BlockSpec((H, H), lambda i: (0, 0)),
        ],
        out_specs=[
            pl.BlockSpec((RB, H), lambda i: (i, 0)),
            pl.BlockSpec((2, H), lambda i: (0, 0)),
        ],
        out_shape=[
            jax.ShapeDtypeStruct((N, H), jnp.float32),
            jax.ShapeDtypeStruct((2, H), jnp.float32),
        ],
        scratch_shapes=[pltpu.VMEM((2, H), jnp.float32)],
    )(agg_lo, agg_hi, h_lo, h_hi, deg, Wl, bl, Wr)


def _tc_layer_b(out, stats, g, b):
    def body(o, st, g_, b_, hlo_ref, hhi_ref):
        mu = st[0:1, :] * (1.0 / N)
        var = st[1:2, :] * (1.0 / N) - mu * mu
        scale = g_[...] * lax.rsqrt(var + EPS)
        shift = b_[...] - mu * scale
        h = jnp.maximum(o[...] * scale + shift, 0.0)
        hlo_ref[...] = h[:, 0:HH]
        hhi_ref[...] = h[:, HH:H]

    return pl.pallas_call(
        body,
        grid=(NRB,),
        in_specs=[
            pl.BlockSpec((RB, H), lambda i: (i, 0)),
            pl.BlockSpec((2, H), lambda i: (0, 0)),
            pl.BlockSpec((1, H), lambda i: (0, 0)),
            pl.BlockSpec((1, H), lambda i: (0, 0)),
        ],
        out_specs=[_row_spec(), _row_spec()],
        out_shape=[
            jax.ShapeDtypeStruct((N, HH), jnp.float32),
            jax.ShapeDtypeStruct((N, HH), jnp.float32),
        ],
    )(out, stats, g, b)


def _tc_final(agg_lo, agg_hi, h_lo, h_hi, deg, Wl, bl, Wr, Wlin, blin):
    def body(al, ah, hl, hh, dg, wl, b, wr, wf, bf, out_ref):
        rdeg = 1.0 / jnp.maximum(dg[...], 1.0)
        ml = al[...] * rdeg
        mh = ah[...] * rdeg
        t = (
            jnp.dot(ml, wl[0:HH, :], preferred_element_type=jnp.float32)
            + jnp.dot(mh, wl[HH:H, :], preferred_element_type=jnp.float32)
            + jnp.dot(hl[...], wr[0:HH, :], preferred_element_type=jnp.float32)
            + jnp.dot(hh[...], wr[HH:H, :], preferred_element_type=jnp.float32)
            + b[...]
        )
        t = jnp.maximum(t, 0.0)
        out_ref[...] = jnp.dot(t, wf[...], preferred_element_type=jnp.float32) + bf[...]

    return pl.pallas_call(
        body,
        grid=(NRB,),
        in_specs=[
            _row_spec(), _row_spec(), _row_spec(), _row_spec(),
            pl.BlockSpec((RB, 1), lambda i: (i, 0)),
            pl.BlockSpec((H, H), lambda i: (0, 0)),
            pl.BlockSpec((1, H), lambda i: (0, 0)),
            pl.BlockSpec((H, H), lambda i: (0, 0)),
            pl.BlockSpec((H, H), lambda i: (0, 0)),
            pl.BlockSpec((1, H), lambda i: (0, 0)),
        ],
        out_specs=pl.BlockSpec((RB, H), lambda i: (i, 0)),
        out_shape=jax.ShapeDtypeStruct((N, H), jnp.float32),
    )(agg_lo, agg_hi, h_lo, h_hi, deg, Wl, bl, Wr, Wlin, blin)


def kernel(x, edge_index, Wl1, bl1, Wr1, g1, b1, Wl2, bl2, Wr2, g2, b2,
           Wl3, bl3, Wr3, Wlin, blin):
    ei = edge_index.astype(jnp.int32)
    src = ei[0]
    dst = ei[1]
    src_r = jnp.concatenate(
        [src.reshape(NS, EPT), jnp.zeros((NS, EPADT), jnp.int32)], axis=1
    ).reshape(NS, NGRP, G, CH)
    dst_r = jnp.concatenate(
        [dst.reshape(NS, EPT), jnp.full((NS, EPADT), N, jnp.int32)], axis=1
    ).reshape(NS, NGRP, G, CH)
    dst_r2 = dst.reshape(2 * NS, DNCH, DCH)
    zeros_lo = jnp.zeros((N, HH), jnp.float32)
    ones40 = jnp.ones((DCH, HH), jnp.float32)
    x_lo = x[:, :HH]
    x_hi = x[:, HH:]

    bl1r, bl2r, bl3r = (v.reshape(1, H) for v in (bl1, bl2, bl3))
    g1r, b1r = g1.reshape(1, H), b1.reshape(1, H)
    g2r, b2r = g2.reshape(1, H), b2.reshape(1, H)
    blinr = blin.reshape(1, H)

    d0, d1 = _sc_degree(dst_r2, zeros_lo, ones40)
    deg = d0[:, :1] + d1[:, :1]

    a1lo, a1hi = _sc_aggregate(x_lo, x_hi, src_r, dst_r, zeros_lo)
    out1, st1 = _tc_layer_a(a1lo, a1hi, x_lo, x_hi, deg, Wl1, bl1r, Wr1)
    h1lo, h1hi = _tc_layer_b(out1, st1, g1r, b1r)

    a2lo, a2hi = _sc_aggregate(h1lo, h1hi, src_r, dst_r, zeros_lo)
    out2, st2 = _tc_layer_a(a2lo, a2hi, h1lo, h1hi, deg, Wl2, bl2r, Wr2)
    h2lo, h2hi = _tc_layer_b(out2, st2, g2r, b2r)

    a3lo, a3hi = _sc_aggregate(h2lo, h2hi, src_r, dst_r, zeros_lo)
    return _tc_final(a3lo, a3hi, h2lo, h2hi, deg, Wl3, bl3r, Wr3, Wlin, blinr)


# speedup vs baseline: 1.6969x; 1.6969x over previous
import functools

import jax
import jax.numpy as jnp
from jax import lax
from jax.experimental import pallas as pl
from jax.experimental.pallas import tpu as pltpu
from jax.experimental.pallas import tpu_sc as plsc

N = 10000
E = 160000
H = 256
HH = 128
EPS = 1e-5

NS = 16
EPT = E // NS
CH = 125
NCH = EPT // CH
G = 16
NGRP = NCH // G
NPAIRG = G // 2
DCH = 125
DNCH = (E // (2 * NS)) // DCH
ROWS_A = 624

_MESH = plsc.VectorSubcoreMesh(core_axis_name="c", subcore_axis_name="s")


def _sc_aggregate(h_lo, h_hi, src_r, dst_r, zeros_lo):
    out_t = jax.ShapeDtypeStruct((N, HH), jnp.float32)

    @functools.partial(
        pl.kernel,
        out_type=(out_t, out_t),
        mesh=_MESH,
        scratch_types=[
            pltpu.VMEM_SHARED((N, HH), jnp.float32),
            pltpu.VMEM((G, CH), jnp.int32),
            pltpu.VMEM((G, CH), jnp.int32),
            pltpu.VMEM((G, CH), jnp.int32),
            pltpu.VMEM((G, CH), jnp.int32),
            pltpu.VMEM((CH, HH), jnp.float32),
            pltpu.VMEM((CH, HH), jnp.float32),
            pltpu.SemaphoreType.DMA,
            pltpu.SemaphoreType.DMA,
            pltpu.SemaphoreType.DMA,
            pltpu.SemaphoreType.DMA,
            pltpu.SemaphoreType.DMA,
            pltpu.SemaphoreType.DMA,
        ],
    )
    def k(hlo, hhi, srcr, dstr, zlo, olo, ohi, acc, si0, si1, di0, di1,
          rows0, rows1, gsem0, gsem1, ssem0, ssem1, isem_s, isem_d):
        c = lax.axis_index("c")
        s = lax.axis_index("s")
        base = s * ROWS_A
        pltpu.sync_copy(zlo.at[pl.ds(base, ROWS_A)], acc.at[pl.ds(base, ROWS_A)])

        @pl.when(s == NS - 1)
        def _():
            pltpu.sync_copy(zlo.at[pl.ds(N - 16, 16)], acc.at[pl.ds(N - 16, 16)])

        pltpu.sync_copy(srcr.at[s, 0], si0)
        pltpu.sync_copy(dstr.at[s, 0], di0)
        plsc.subcore_barrier()

        def edge_loop(h_src):
            def g_start(ib, l, buf, sem_):
                pltpu.make_async_copy(h_src.at[ib.at[l]], buf, sem_).start()

            def g_wait(ib, l, buf, sem_):
                pltpu.make_async_copy(h_src.at[ib.at[l]], buf, sem_).wait()

            def s_start(ib, l, buf, sem_):
                pltpu.make_async_copy(buf, acc.at[ib.at[l]], sem_).start(add=True)

            def s_wait(ib, buf, sem_):
                pltpu.make_async_copy(buf, acc.at[ib.at[0]], sem_).wait()

            def do_pair(sib, dib, l0, is_first):
                l1 = l0 + 1
                g_wait(sib, l0, rows0, gsem0)
                if not is_first:
                    s_wait(dib, rows1, ssem1)
                g_start(sib, l1, rows1, gsem1)
                s_start(dib, l0, rows0, ssem0)
                g_wait(sib, l1, rows1, gsem1)
                s_wait(dib, rows0, ssem0)
                s_start(dib, l1, rows1, ssem1)

            g_start(si0, 0, rows0, gsem0)

            for grp in range(NGRP):
                sib, dib = (si0, di0) if grp % 2 == 0 else (si1, di1)
                nsib, ndib = (si1, di1) if grp % 2 == 0 else (si0, di0)
                last_grp = grp == NGRP - 1

                do_pair(sib, dib, 0, is_first=(grp == 0))
                if not last_grp:
                    pltpu.make_async_copy(srcr.at[s, grp + 1], nsib, isem_s).start()
                    pltpu.make_async_copy(dstr.at[s, grp + 1], ndib, isem_d).start()
                g_start(sib, 2, rows0, gsem0)

                @pl.loop(1, NPAIRG - 1)
                def _(t):
                    do_pair(sib, dib, 2 * t, False)
                    g_start(sib, 2 * t + 2, rows0, gsem0)

                do_pair(sib, dib, G - 2, False)
                if last_grp:
                    s_wait(dib, rows1, ssem1)
                else:
                    pltpu.make_async_copy(srcr.at[s, grp + 1], nsib, isem_s).wait()
                    pltpu.make_async_copy(dstr.at[s, grp + 1], ndib, isem_d).wait()
                    g_start(nsib, 0, rows0, gsem0)

        @pl.when(c == 0)
        def _():
            edge_loop(hlo)

        @pl.when(c == 1)
        def _():
            edge_loop(hhi)

        plsc.subcore_barrier()

        def writeout(o):
            pltpu.sync_copy(acc.at[pl.ds(base, ROWS_A)], o.at[pl.ds(base, ROWS_A)])

            @pl.when(s == NS - 1)
            def _():
                pltpu.sync_copy(acc.at[pl.ds(N - 16, 16)], o.at[pl.ds(N - 16, 16)])

        @pl.when(c == 0)
        def _():
            writeout(olo)

        @pl.when(c == 1)
        def _():
            writeout(ohi)

    return k(h_lo, h_hi, src_r, dst_r, zeros_lo)


def _sc_degree(dst_r2, zeros_lo, ones40):
    out_t = jax.ShapeDtypeStruct((N, HH), jnp.float32)

    @functools.partial(
        pl.kernel,
        out_type=(out_t, out_t),
        mesh=_MESH,
        scratch_types=[
            pltpu.VMEM_SHARED((N, HH), jnp.float32),
            pltpu.VMEM((DNCH, DCH), jnp.int32),
            pltpu.VMEM((DCH, HH), jnp.float32),
            pltpu.SemaphoreType.DMA,
        ],
    )
    def k(dstr, zlo, ones_hbm, o0, o1, acc, didx, ones, ssem):
        c = lax.axis_index("c")
        s = lax.axis_index("s")
        base = s * ROWS_A
        pltpu.sync_copy(zlo.at[pl.ds(base, ROWS_A)], acc.at[pl.ds(base, ROWS_A)])

        @pl.when(s == NS - 1)
        def _():
            pltpu.sync_copy(zlo.at[pl.ds(N - 16, 16)], acc.at[pl.ds(N - 16, 16)])

        tid = c * NS + s
        pltpu.sync_copy(dstr.at[tid], didx)
        pltpu.sync_copy(ones_hbm, ones)
        plsc.subcore_barrier()

        @pl.loop(0, DNCH)
        def _(j):
            pltpu.make_async_copy(ones, acc.at[didx.at[j]], ssem).start(add=True)

        @pl.loop(0, DNCH)
        def _(j):
            pltpu.make_async_copy(ones, acc.at[didx.at[0]], ssem).wait()

        plsc.subcore_barrier()

        def writeout(o):
            pltpu.sync_copy(acc.at[pl.ds(base, ROWS_A)], o.at[pl.ds(base, ROWS_A)])

            @pl.when(s == NS - 1)
            def _():
                pltpu.sync_copy(acc.at[pl.ds(N - 16, 16)], o.at[pl.ds(N - 16, 16)])

        @pl.when(c == 0)
        def _():
            writeout(o0)

        @pl.when(c == 1)
        def _():
            writeout(o1)

    return k(dst_r2, zeros_lo, ones40)


RB = 1000
NRB = N // RB


def _row_spec():
    return pl.BlockSpec((RB, HH), lambda i: (i, 0))


def _tc_layer_a(agg_lo, agg_hi, h_lo, h_hi, deg, Wl, bl, Wr):
    def body(al, ah, hl, hh, dg, wl, b, wr, out_ref, st_ref, accs):
        i = pl.program_id(0)
        rdeg = 1.0 / jnp.maximum(dg[...], 1.0)
        ml = al[...] * rdeg
        mh = ah[...] * rdeg
        out = (
            jnp.dot(ml, wl[0:HH, :], preferred_element_type=jnp.float32)
            + jnp.dot(mh, wl[HH:H, :], preferred_element_type=jnp.float32)
            + jnp.dot(hl[...], wr[0:HH, :], preferred_element_type=jnp.float32)
            + jnp.dot(hh[...], wr[HH:H, :], preferred_element_type=jnp.float32)
            + b[...]
        )
        out_ref[...] = out

        @pl.when(i == 0)
        def _():
            accs[...] = jnp.zeros_like(accs)

        accs[0:1, :] += jnp.sum(out, axis=0, keepdims=True)
        accs[1:2, :] += jnp.sum(out * out, axis=0, keepdims=True)

        @pl.when(i == NRB - 1)
        def _():
            st_ref[...] = accs[...]

    return pl.pallas_call(
        body,
        grid=(NRB,),
        in_specs=[
            _row_spec(), _row_spec(), _row_spec(), _row_spec(),
            pl.BlockSpec((RB, 1), lambda i: (i, 0)),
            pl.BlockSpec((H, H), lambda i: (0, 0)),
            pl.BlockSpec((1, H), lambda i: (0, 0)),
            pl.BlockSpec((H, H), lambda i: (0, 0)),
        ],
        out_specs=[
            pl.BlockSpec((RB, H), lambda i: (i, 0)),
            pl.BlockSpec((2, H), lambda i: (0, 0)),
        ],
        out_shape=[
            jax.ShapeDtypeStruct((N, H), jnp.float32),
            jax.ShapeDtypeStruct((2, H), jnp.float32),
        ],
        scratch_shapes=[pltpu.VMEM((2, H), jnp.float32)],
    )(agg_lo, agg_hi, h_lo, h_hi, deg, Wl, bl, Wr)


def _tc_layer_b(out, stats, g, b):
    def body(o, st, g_, b_, hlo_ref, hhi_ref):
        mu = st[0:1, :] * (1.0 / N)
        var = st[1:2, :] * (1.0 / N) - mu * mu
        scale = g_[...] * lax.rsqrt(var + EPS)
        shift = b_[...] - mu * scale
        h = jnp.maximum(o[...] * scale + shift, 0.0)
        hlo_ref[...] = h[:, 0:HH]
        hhi_ref[...] = h[:, HH:H]

    return pl.pallas_call(
        body,
        grid=(NRB,),
        in_specs=[
            pl.BlockSpec((RB, H), lambda i: (i, 0)),
            pl.BlockSpec((2, H), lambda i: (0, 0)),
            pl.BlockSpec((1, H), lambda i: (0, 0)),
            pl.BlockSpec((1, H), lambda i: (0, 0)),
        ],
        out_specs=[_row_spec(), _row_spec()],
        out_shape=[
            jax.ShapeDtypeStruct((N, HH), jnp.float32),
            jax.ShapeDtypeStruct((N, HH), jnp.float32),
        ],
    )(out, stats, g, b)


def _tc_final(agg_lo, agg_hi, h_lo, h_hi, deg, Wl, bl, Wr, Wlin, blin):
    def body(al, ah, hl, hh, dg, wl, b, wr, wf, bf, out_ref):
        rdeg = 1.0 / jnp.maximum(dg[...], 1.0)
        ml = al[...] * rdeg
        mh = ah[...] * rdeg
        t = (
            jnp.dot(ml, wl[0:HH, :], preferred_element_type=jnp.float32)
            + jnp.dot(mh, wl[HH:H, :], preferred_element_type=jnp.float32)
            + jnp.dot(hl[...], wr[0:HH, :], preferred_element_type=jnp.float32)
            + jnp.dot(hh[...], wr[HH:H, :], preferred_element_type=jnp.float32)
            + b[...]
        )
        t = jnp.maximum(t, 0.0)
        out_ref[...] = jnp.dot(t, wf[...], preferred_element_type=jnp.float32) + bf[...]

    return pl.pallas_call(
        body,
        grid=(NRB,),
        in_specs=[
            _row_spec(), _row_spec(), _row_spec(), _row_spec(),
            pl.BlockSpec((RB, 1), lambda i: (i, 0)),
            pl.BlockSpec((H, H), lambda i: (0, 0)),
            pl.BlockSpec((1, H), lambda i: (0, 0)),
            pl.BlockSpec((H, H), lambda i: (0, 0)),
            pl.BlockSpec((H, H), lambda i: (0, 0)),
            pl.BlockSpec((1, H), lambda i: (0, 0)),
        ],
        out_specs=pl.BlockSpec((RB, H), lambda i: (i, 0)),
        out_shape=jax.ShapeDtypeStruct((N, H), jnp.float32),
    )(agg_lo, agg_hi, h_lo, h_hi, deg, Wl, bl, Wr, Wlin, blin)


def kernel(x, edge_index, Wl1, bl1, Wr1, g1, b1, Wl2, bl2, Wr2, g2, b2,
           Wl3, bl3, Wr3, Wlin, blin):
    ei = edge_index.astype(jnp.int32)
    src = ei[0]
    dst = ei[1]
    src_r = src.reshape(NS, NGRP, G, CH)
    dst_r = dst.reshape(NS, NGRP, G, CH)
    dst_r2 = dst.reshape(2 * NS, DNCH, DCH)
    zeros_lo = jnp.zeros((N, HH), jnp.float32)
    ones40 = jnp.ones((DCH, HH), jnp.float32)
    x_lo = x[:, :HH]
    x_hi = x[:, HH:]

    bl1r, bl2r, bl3r = (v.reshape(1, H) for v in (bl1, bl2, bl3))
    g1r, b1r = g1.reshape(1, H), b1.reshape(1, H)
    g2r, b2r = g2.reshape(1, H), b2.reshape(1, H)
    blinr = blin.reshape(1, H)

    d0, d1 = _sc_degree(dst_r2, zeros_lo, ones40)
    deg = d0[:, :1] + d1[:, :1]

    a1lo, a1hi = _sc_aggregate(x_lo, x_hi, src_r, dst_r, zeros_lo)
    out1, st1 = _tc_layer_a(a1lo, a1hi, x_lo, x_hi, deg, Wl1, bl1r, Wr1)
    h1lo, h1hi = _tc_layer_b(out1, st1, g1r, b1r)

    a2lo, a2hi = _sc_aggregate(h1lo, h1hi, src_r, dst_r, zeros_lo)
    out2, st2 = _tc_layer_a(a2lo, a2hi, h1lo, h1hi, deg, Wl2, bl2r, Wr2)
    h2lo, h2hi = _tc_layer_b(out2, st2, g2r, b2r)

    a3lo, a3hi = _sc_aggregate(h2lo, h2hi, src_r, dst_r, zeros_lo)
    return _tc_final(a3lo, a3hi, h2lo, h2hi, deg, Wl3, bl3r, Wr3, Wlin, blinr)


# speedup vs baseline: 1.7435x; 1.0274x over previous
import functools

import jax
import jax.numpy as jnp
from jax import lax
from jax.experimental import pallas as pl
from jax.experimental.pallas import tpu as pltpu
from jax.experimental.pallas import tpu_sc as plsc

N = 10000
E = 160000
H = 256
HH = 128
EPS = 1e-5

NS = 16
EPT = E // NS
CH = 125
NCH = EPT // CH
G = 16
NGRP = NCH // G
NPAIRG = G // 2
DCH = 125
DNCH = (E // (2 * NS)) // DCH
ROWS_A = 624

_MESH = plsc.VectorSubcoreMesh(core_axis_name="c", subcore_axis_name="s")


def _sc_aggregate(h_lo, h_hi, src_r, dst_r, zeros_lo):
    out_t = jax.ShapeDtypeStruct((N, HH), jnp.float32)

    @functools.partial(
        pl.kernel,
        out_type=(out_t, out_t),
        mesh=_MESH,
        scratch_types=[
            pltpu.VMEM_SHARED((N, HH), jnp.float32),
            pltpu.VMEM((G, CH), jnp.int32),
            pltpu.VMEM((G, CH), jnp.int32),
            pltpu.VMEM((G, CH), jnp.int32),
            pltpu.VMEM((G, CH), jnp.int32),
            pltpu.VMEM((CH, HH), jnp.float32),
            pltpu.VMEM((CH, HH), jnp.float32),
            pltpu.SemaphoreType.DMA,
            pltpu.SemaphoreType.DMA,
            pltpu.SemaphoreType.DMA,
            pltpu.SemaphoreType.DMA,
            pltpu.SemaphoreType.DMA,
            pltpu.SemaphoreType.DMA,
        ],
    )
    def k(hlo, hhi, srcr, dstr, zlo, olo, ohi, acc, si0, si1, di0, di1,
          rows0, rows1, gsem0, gsem1, ssem0, ssem1, isem_s, isem_d):
        c = lax.axis_index("c")
        s = lax.axis_index("s")
        base = s * ROWS_A
        pltpu.sync_copy(zlo.at[pl.ds(base, ROWS_A)], acc.at[pl.ds(base, ROWS_A)])

        @pl.when(s == NS - 1)
        def _():
            pltpu.sync_copy(zlo.at[pl.ds(N - 16, 16)], acc.at[pl.ds(N - 16, 16)])

        pltpu.sync_copy(srcr.at[s, 0], si0)
        pltpu.sync_copy(dstr.at[s, 0], di0)
        plsc.subcore_barrier()

        def edge_loop(h_src):
            def g_start(ib, l, buf, sem_):
                pltpu.make_async_copy(h_src.at[ib.at[l]], buf, sem_).start()

            def g_wait(ib, l, buf, sem_):
                pltpu.make_async_copy(h_src.at[ib.at[l]], buf, sem_).wait()

            def s_start(ib, l, buf, sem_):
                pltpu.make_async_copy(buf, acc.at[ib.at[l]], sem_).start(add=True)

            def s_wait(ib, buf, sem_):
                pltpu.make_async_copy(buf, acc.at[ib.at[0]], sem_).wait()

            def do_pair(sib, dib, l0, is_first):
                l1 = l0 + 1
                g_wait(sib, l0, rows0, gsem0)
                if not is_first:
                    s_wait(dib, rows1, ssem1)
                g_start(sib, l1, rows1, gsem1)
                s_start(dib, l0, rows0, ssem0)
                g_wait(sib, l1, rows1, gsem1)
                s_wait(dib, rows0, ssem0)
                s_start(dib, l1, rows1, ssem1)

            g_start(si0, 0, rows0, gsem0)

            for grp in range(NGRP):
                sib, dib = (si0, di0) if grp % 2 == 0 else (si1, di1)
                nsib, ndib = (si1, di1) if grp % 2 == 0 else (si0, di0)
                last_grp = grp == NGRP - 1

                do_pair(sib, dib, 0, is_first=(grp == 0))
                if not last_grp:
                    pltpu.make_async_copy(srcr.at[s, grp + 1], nsib, isem_s).start()
                    pltpu.make_async_copy(dstr.at[s, grp + 1], ndib, isem_d).start()
                g_start(sib, 2, rows0, gsem0)

                @pl.loop(1, NPAIRG - 1)
                def _(t):
                    do_pair(sib, dib, 2 * t, False)
                    g_start(sib, 2 * t + 2, rows0, gsem0)

                do_pair(sib, dib, G - 2, False)
                if last_grp:
                    s_wait(dib, rows1, ssem1)
                else:
                    pltpu.make_async_copy(srcr.at[s, grp + 1], nsib, isem_s).wait()
                    pltpu.make_async_copy(dstr.at[s, grp + 1], ndib, isem_d).wait()
                    g_start(nsib, 0, rows0, gsem0)

        @pl.when(c == 0)
        def _():
            edge_loop(hlo)

        @pl.when(c == 1)
        def _():
            edge_loop(hhi)

        plsc.subcore_barrier()

        def writeout(o):
            pltpu.sync_copy(acc.at[pl.ds(base, ROWS_A)], o.at[pl.ds(base, ROWS_A)])

            @pl.when(s == NS - 1)
            def _():
                pltpu.sync_copy(acc.at[pl.ds(N - 16, 16)], o.at[pl.ds(N - 16, 16)])

        @pl.when(c == 0)
        def _():
            writeout(olo)

        @pl.when(c == 1)
        def _():
            writeout(ohi)

    return k(h_lo, h_hi, src_r, dst_r, zeros_lo)


def _sc_degree(dst_r2, zeros_lo, ones40):
    out_t = jax.ShapeDtypeStruct((N, HH), jnp.float32)

    @functools.partial(
        pl.kernel,
        out_type=(out_t, out_t),
        mesh=_MESH,
        scratch_types=[
            pltpu.VMEM_SHARED((N, HH), jnp.float32),
            pltpu.VMEM((DNCH, DCH), jnp.int32),
            pltpu.VMEM((DCH, HH), jnp.float32),
            pltpu.SemaphoreType.DMA,
        ],
    )
    def k(dstr, zlo, ones_hbm, o0, o1, acc, didx, ones, ssem):
        c = lax.axis_index("c")
        s = lax.axis_index("s")
        base = s * ROWS_A
        pltpu.sync_copy(zlo.at[pl.ds(base, ROWS_A)], acc.at[pl.ds(base, ROWS_A)])

        @pl.when(s == NS - 1)
        def _():
            pltpu.sync_copy(zlo.at[pl.ds(N - 16, 16)], acc.at[pl.ds(N - 16, 16)])

        tid = c * NS + s
        pltpu.sync_copy(dstr.at[tid], didx)
        pltpu.sync_copy(ones_hbm, ones)
        plsc.subcore_barrier()

        @pl.loop(0, DNCH)
        def _(j):
            pltpu.make_async_copy(ones, acc.at[didx.at[j]], ssem).start(add=True)

        @pl.loop(0, DNCH)
        def _(j):
            pltpu.make_async_copy(ones, acc.at[didx.at[0]], ssem).wait()

        plsc.subcore_barrier()

        def writeout(o):
            pltpu.sync_copy(acc.at[pl.ds(base, ROWS_A)], o.at[pl.ds(base, ROWS_A)])

            @pl.when(s == NS - 1)
            def _():
                pltpu.sync_copy(acc.at[pl.ds(N - 16, 16)], o.at[pl.ds(N - 16, 16)])

        @pl.when(c == 0)
        def _():
            writeout(o0)

        @pl.when(c == 1)
        def _():
            writeout(o1)

    return k(dst_r2, zeros_lo, ones40)


RB = 1000
NRB = N // RB


def _row_spec():
    return pl.BlockSpec((RB, HH), lambda i: (i, 0))


def _dot(a, w):
    return jnp.dot(a.astype(jnp.bfloat16), w.astype(jnp.bfloat16),
                   preferred_element_type=jnp.float32)


def _tc_layer(agg_lo, agg_hi, h_lo, h_hi, deg, Wl, bl, Wr, g, b):
    def body(al, ah, hl, hh, dg, wl, b_, wr, g_, be, hlo_ref, hhi_ref,
             obuf, accs):
        p = pl.program_id(0)
        i = pl.program_id(1)

        @pl.when(p == 0)
        def _():
            rdeg = 1.0 / jnp.maximum(dg[...], 1.0)
            out = (
                _dot(al[...] * rdeg, wl[0:HH, :])
                + _dot(ah[...] * rdeg, wl[HH:H, :])
                + _dot(hl[...], wr[0:HH, :])
                + _dot(hh[...], wr[HH:H, :])
                + b_[...]
            )
            obuf[pl.ds(i * RB, RB), :] = out

            @pl.when(i == 0)
            def _():
                accs[...] = jnp.zeros_like(accs)

            accs[0:1, :] += jnp.sum(out, axis=0, keepdims=True)
            accs[1:2, :] += jnp.sum(out * out, axis=0, keepdims=True)

        @pl.when(p == 1)
        def _():
            mu = accs[0:1, :] * (1.0 / N)
            var = accs[1:2, :] * (1.0 / N) - mu * mu
            scale = g_[...] * lax.rsqrt(var + EPS)
            shift = be[...] - mu * scale
            h = jnp.maximum(obuf[pl.ds(i * RB, RB), :] * scale + shift, 0.0)
            hlo_ref[...] = h[:, 0:HH]
            hhi_ref[...] = h[:, HH:H]

    def _rs():
        return pl.BlockSpec((RB, HH), lambda p, i: (i * (1 - p), 0))

    def _const(shape):
        return pl.BlockSpec(shape, lambda p, i: (0, 0))

    return pl.pallas_call(
        body,
        grid=(2, NRB),
        in_specs=[
            _rs(), _rs(), _rs(), _rs(),
            pl.BlockSpec((RB, 1), lambda p, i: (i * (1 - p), 0)),
            _const((H, H)), _const((1, H)), _const((H, H)),
            _const((1, H)), _const((1, H)),
        ],
        out_specs=[
            pl.BlockSpec((RB, HH), lambda p, i: (i * p, 0)),
            pl.BlockSpec((RB, HH), lambda p, i: (i * p, 0)),
        ],
        out_shape=[
            jax.ShapeDtypeStruct((N, HH), jnp.float32),
            jax.ShapeDtypeStruct((N, HH), jnp.float32),
        ],
        scratch_shapes=[
            pltpu.VMEM((N, H), jnp.float32),
            pltpu.VMEM((2, H), jnp.float32),
        ],
    )(agg_lo, agg_hi, h_lo, h_hi, deg, Wl, bl, Wr, g, b)


def _tc_final(agg_lo, agg_hi, h_lo, h_hi, deg, Wl, bl, Wr, Wlin, blin):
    def body(al, ah, hl, hh, dg, wl, b, wr, wf, bf, out_ref):
        rdeg = 1.0 / jnp.maximum(dg[...], 1.0)
        t = (
            _dot(al[...] * rdeg, wl[0:HH, :])
            + _dot(ah[...] * rdeg, wl[HH:H, :])
            + _dot(hl[...], wr[0:HH, :])
            + _dot(hh[...], wr[HH:H, :])
            + b[...]
        )
        t = jnp.maximum(t, 0.0)
        out_ref[...] = _dot(t, wf[...]) + bf[...]

    return pl.pallas_call(
        body,
        grid=(NRB,),
        in_specs=[
            _row_spec(), _row_spec(), _row_spec(), _row_spec(),
            pl.BlockSpec((RB, 1), lambda i: (i, 0)),
            pl.BlockSpec((H, H), lambda i: (0, 0)),
            pl.BlockSpec((1, H), lambda i: (0, 0)),
            pl.BlockSpec((H, H), lambda i: (0, 0)),
            pl.BlockSpec((H, H), lambda i: (0, 0)),
            pl.BlockSpec((1, H), lambda i: (0, 0)),
        ],
        out_specs=pl.BlockSpec((RB, H), lambda i: (i, 0)),
        out_shape=jax.ShapeDtypeStruct((N, H), jnp.float32),
    )(agg_lo, agg_hi, h_lo, h_hi, deg, Wl, bl, Wr, Wlin, blin)


def kernel(x, edge_index, Wl1, bl1, Wr1, g1, b1, Wl2, bl2, Wr2, g2, b2,
           Wl3, bl3, Wr3, Wlin, blin):
    ei = edge_index.astype(jnp.int32)
    src = ei[0]
    dst = ei[1]
    src_r = src.reshape(NS, NGRP, G, CH)
    dst_r = dst.reshape(NS, NGRP, G, CH)
    dst_r2 = dst.reshape(2 * NS, DNCH, DCH)
    zeros_lo = jnp.zeros((N, HH), jnp.float32)
    ones40 = jnp.ones((DCH, HH), jnp.float32)
    x_lo = x[:, :HH]
    x_hi = x[:, HH:]

    bl1r, bl2r, bl3r = (v.reshape(1, H) for v in (bl1, bl2, bl3))
    g1r, b1r = g1.reshape(1, H), b1.reshape(1, H)
    g2r, b2r = g2.reshape(1, H), b2.reshape(1, H)
    blinr = blin.reshape(1, H)

    d0, d1 = _sc_degree(dst_r2, zeros_lo, ones40)
    deg = d0[:, :1] + d1[:, :1]

    a1lo, a1hi = _sc_aggregate(x_lo, x_hi, src_r, dst_r, zeros_lo)
    h1lo, h1hi = _tc_layer(a1lo, a1hi, x_lo, x_hi, deg, Wl1, bl1r, Wr1, g1r, b1r)

    a2lo, a2hi = _sc_aggregate(h1lo, h1hi, src_r, dst_r, zeros_lo)
    h2lo, h2hi = _tc_layer(a2lo, a2hi, h1lo, h1hi, deg, Wl2, bl2r, Wr2, g2r, b2r)

    a3lo, a3hi = _sc_aggregate(h2lo, h2hi, src_r, dst_r, zeros_lo)
    return _tc_final(a3lo, a3hi, h2lo, h2hi, deg, Wl3, bl3r, Wr3, Wlin, blinr)


# speedup vs baseline: 1.7478x; 1.0025x over previous
import functools

import jax
import jax.numpy as jnp
from jax import lax
from jax.experimental import pallas as pl
from jax.experimental.pallas import tpu as pltpu
from jax.experimental.pallas import tpu_sc as plsc

N = 10000
E = 160000
H = 256
HH = 128
EPS = 1e-5

NS = 16
EPT = E // NS
CH = 125
NCH = EPT // CH
G = 16
NGRP = NCH // G
NPAIRG = G // 2
DCH = 125
DNCH = (E // (2 * NS)) // DCH
ROWS_A = 624

_MESH = plsc.VectorSubcoreMesh(core_axis_name="c", subcore_axis_name="s")


def _sc_aggregate(h_lo, h_hi, src_r, dst_r, zeros_lo):
    out_t = jax.ShapeDtypeStruct((N, HH), jnp.float32)

    @functools.partial(
        pl.kernel,
        out_type=(out_t, out_t),
        mesh=_MESH,
        scratch_types=[
            pltpu.VMEM_SHARED((N, HH), jnp.float32),
            pltpu.VMEM((G, CH), jnp.int32),
            pltpu.VMEM((G, CH), jnp.int32),
            pltpu.VMEM((G, CH), jnp.int32),
            pltpu.VMEM((G, CH), jnp.int32),
            pltpu.VMEM((CH, HH), jnp.float32),
            pltpu.VMEM((CH, HH), jnp.float32),
            pltpu.SemaphoreType.DMA,
            pltpu.SemaphoreType.DMA,
            pltpu.SemaphoreType.DMA,
            pltpu.SemaphoreType.DMA,
            pltpu.SemaphoreType.DMA,
            pltpu.SemaphoreType.DMA,
        ],
    )
    def k(hlo, hhi, srcr, dstr, zlo, olo, ohi, acc, si0, si1, di0, di1,
          rows0, rows1, gsem0, gsem1, ssem0, ssem1, isem_s, isem_d):
        c = lax.axis_index("c")
        s = lax.axis_index("s")
        base = s * ROWS_A
        pltpu.sync_copy(zlo.at[pl.ds(base, ROWS_A)], acc.at[pl.ds(base, ROWS_A)])

        @pl.when(s == NS - 1)
        def _():
            pltpu.sync_copy(zlo.at[pl.ds(N - 16, 16)], acc.at[pl.ds(N - 16, 16)])

        pltpu.sync_copy(srcr.at[s, 0], si0)
        pltpu.sync_copy(dstr.at[s, 0], di0)
        plsc.subcore_barrier()

        def edge_loop(h_src):
            def g_start(ib, l, buf, sem_):
                pltpu.make_async_copy(h_src.at[ib.at[l]], buf, sem_).start()

            def g_wait(ib, l, buf, sem_):
                pltpu.make_async_copy(h_src.at[ib.at[l]], buf, sem_).wait()

            def s_start(ib, l, buf, sem_):
                pltpu.make_async_copy(buf, acc.at[ib.at[l]], sem_).start(add=True)

            def s_wait(ib, buf, sem_):
                pltpu.make_async_copy(buf, acc.at[ib.at[0]], sem_).wait()

            def do_pair(sib, dib, l0, is_first):
                l1 = l0 + 1
                g_wait(sib, l0, rows0, gsem0)
                if not is_first:
                    s_wait(dib, rows1, ssem1)
                g_start(sib, l1, rows1, gsem1)
                s_start(dib, l0, rows0, ssem0)
                g_wait(sib, l1, rows1, gsem1)
                s_start(dib, l1, rows1, ssem1)
                s_wait(dib, rows0, ssem0)

            g_start(si0, 0, rows0, gsem0)

            for grp in range(NGRP):
                sib, dib = (si0, di0) if grp % 2 == 0 else (si1, di1)
                nsib, ndib = (si1, di1) if grp % 2 == 0 else (si0, di0)
                last_grp = grp == NGRP - 1

                do_pair(sib, dib, 0, is_first=(grp == 0))
                if not last_grp:
                    pltpu.make_async_copy(srcr.at[s, grp + 1], nsib, isem_s).start()
                    pltpu.make_async_copy(dstr.at[s, grp + 1], ndib, isem_d).start()
                g_start(sib, 2, rows0, gsem0)

                @pl.loop(1, NPAIRG - 1)
                def _(t):
                    do_pair(sib, dib, 2 * t, False)
                    g_start(sib, 2 * t + 2, rows0, gsem0)

                do_pair(sib, dib, G - 2, False)
                if last_grp:
                    s_wait(dib, rows1, ssem1)
                else:
                    pltpu.make_async_copy(srcr.at[s, grp + 1], nsib, isem_s).wait()
                    pltpu.make_async_copy(dstr.at[s, grp + 1], ndib, isem_d).wait()
                    g_start(nsib, 0, rows0, gsem0)

        @pl.when(c == 0)
        def _():
            edge_loop(hlo)

        @pl.when(c == 1)
        def _():
            edge_loop(hhi)

        plsc.subcore_barrier()

        def writeout(o):
            pltpu.sync_copy(acc.at[pl.ds(base, ROWS_A)], o.at[pl.ds(base, ROWS_A)])

            @pl.when(s == NS - 1)
            def _():
                pltpu.sync_copy(acc.at[pl.ds(N - 16, 16)], o.at[pl.ds(N - 16, 16)])

        @pl.when(c == 0)
        def _():
            writeout(olo)

        @pl.when(c == 1)
        def _():
            writeout(ohi)

    return k(h_lo, h_hi, src_r, dst_r, zeros_lo)


def _sc_degree(dst_r2, zeros_lo, ones40):
    out_t = jax.ShapeDtypeStruct((N, HH), jnp.float32)

    @functools.partial(
        pl.kernel,
        out_type=(out_t, out_t),
        mesh=_MESH,
        scratch_types=[
            pltpu.VMEM_SHARED((N, HH), jnp.float32),
            pltpu.VMEM((DNCH, DCH), jnp.int32),
            pltpu.VMEM((DCH, HH), jnp.float32),
            pltpu.SemaphoreType.DMA,
        ],
    )
    def k(dstr, zlo, ones_hbm, o0, o1, acc, didx, ones, ssem):
        c = lax.axis_index("c")
        s = lax.axis_index("s")
        base = s * ROWS_A
        pltpu.sync_copy(zlo.at[pl.ds(base, ROWS_A)], acc.at[pl.ds(base, ROWS_A)])

        @pl.when(s == NS - 1)
        def _():
            pltpu.sync_copy(zlo.at[pl.ds(N - 16, 16)], acc.at[pl.ds(N - 16, 16)])

        tid = c * NS + s
        pltpu.sync_copy(dstr.at[tid], didx)
        pltpu.sync_copy(ones_hbm, ones)
        plsc.subcore_barrier()

        @pl.loop(0, DNCH)
        def _(j):
            pltpu.make_async_copy(ones, acc.at[didx.at[j]], ssem).start(add=True)

        @pl.loop(0, DNCH)
        def _(j):
            pltpu.make_async_copy(ones, acc.at[didx.at[0]], ssem).wait()

        plsc.subcore_barrier()

        def writeout(o):
            pltpu.sync_copy(acc.at[pl.ds(base, ROWS_A)], o.at[pl.ds(base, ROWS_A)])

            @pl.when(s == NS - 1)
            def _():
                pltpu.sync_copy(acc.at[pl.ds(N - 16, 16)], o.at[pl.ds(N - 16, 16)])

        @pl.when(c == 0)
        def _():
            writeout(o0)

        @pl.when(c == 1)
        def _():
            writeout(o1)

    return k(dst_r2, zeros_lo, ones40)


RB = 1000
NRB = N // RB


def _row_spec():
    return pl.BlockSpec((RB, HH), lambda i: (i, 0))


def _dot(a, w):
    return jnp.dot(a.astype(jnp.bfloat16), w.astype(jnp.bfloat16),
                   preferred_element_type=jnp.float32)


def _tc_layer(agg_lo, agg_hi, h_lo, h_hi, deg, Wl, bl, Wr, g, b):
    def body(al, ah, hl, hh, dg, wl, b_, wr, g_, be, hlo_ref, hhi_ref,
             obuf, accs):
        p = pl.program_id(0)
        i = pl.program_id(1)

        @pl.when(p == 0)
        def _():
            rdeg = 1.0 / jnp.maximum(dg[...], 1.0)
            out = (
                _dot(al[...] * rdeg, wl[0:HH, :])
                + _dot(ah[...] * rdeg, wl[HH:H, :])
                + _dot(hl[...], wr[0:HH, :])
                + _dot(hh[...], wr[HH:H, :])
                + b_[...]
            )
            obuf[pl.ds(i * RB, RB), :] = out

            @pl.when(i == 0)
            def _():
                accs[...] = jnp.zeros_like(accs)

            accs[0:1, :] += jnp.sum(out, axis=0, keepdims=True)
            accs[1:2, :] += jnp.sum(out * out, axis=0, keepdims=True)

        @pl.when(p == 1)
        def _():
            mu = accs[0:1, :] * (1.0 / N)
            var = accs[1:2, :] * (1.0 / N) - mu * mu
            scale = g_[...] * lax.rsqrt(var + EPS)
            shift = be[...] - mu * scale
            h = jnp.maximum(obuf[pl.ds(i * RB, RB), :] * scale + shift, 0.0)
            hlo_ref[...] = h[:, 0:HH]
            hhi_ref[...] = h[:, HH:H]

    def _rs():
        return pl.BlockSpec((RB, HH), lambda p, i: (i * (1 - p), 0))

    def _const(shape):
        return pl.BlockSpec(shape, lambda p, i: (0, 0))

    return pl.pallas_call(
        body,
        grid=(2, NRB),
        in_specs=[
            _rs(), _rs(), _rs(), _rs(),
            pl.BlockSpec((RB, 1), lambda p, i: (i * (1 - p), 0)),
            _const((H, H)), _const((1, H)), _const((H, H)),
            _const((1, H)), _const((1, H)),
        ],
        out_specs=[
            pl.BlockSpec((RB, HH), lambda p, i: (i * p, 0)),
            pl.BlockSpec((RB, HH), lambda p, i: (i * p, 0)),
        ],
        out_shape=[
            jax.ShapeDtypeStruct((N, HH), jnp.float32),
            jax.ShapeDtypeStruct((N, HH), jnp.float32),
        ],
        scratch_shapes=[
            pltpu.VMEM((N, H), jnp.float32),
            pltpu.VMEM((2, H), jnp.float32),
        ],
    )(agg_lo, agg_hi, h_lo, h_hi, deg, Wl, bl, Wr, g, b)


def _tc_final(agg_lo, agg_hi, h_lo, h_hi, deg, Wl, bl, Wr, Wlin, blin):
    def body(al, ah, hl, hh, dg, wl, b, wr, wf, bf, out_ref):
        rdeg = 1.0 / jnp.maximum(dg[...], 1.0)
        t = (
            _dot(al[...] * rdeg, wl[0:HH, :])
            + _dot(ah[...] * rdeg, wl[HH:H, :])
            + _dot(hl[...], wr[0:HH, :])
            + _dot(hh[...], wr[HH:H, :])
            + b[...]
        )
        t = jnp.maximum(t, 0.0)
        out_ref[...] = _dot(t, wf[...]) + bf[...]

    return pl.pallas_call(
        body,
        grid=(NRB,),
        in_specs=[
            _row_spec(), _row_spec(), _row_spec(), _row_spec(),
            pl.BlockSpec((RB, 1), lambda i: (i, 0)),
            pl.BlockSpec((H, H), lambda i: (0, 0)),
            pl.BlockSpec((1, H), lambda i: (0, 0)),
            pl.BlockSpec((H, H), lambda i: (0, 0)),
            pl.BlockSpec((H, H), lambda i: (0, 0)),
            pl.BlockSpec((1, H), lambda i: (0, 0)),
        ],
        out_specs=pl.BlockSpec((RB, H), lambda i: (i, 0)),
        out_shape=jax.ShapeDtypeStruct((N, H), jnp.float32),
    )(agg_lo, agg_hi, h_lo, h_hi, deg, Wl, bl, Wr, Wlin, blin)


def kernel(x, edge_index, Wl1, bl1, Wr1, g1, b1, Wl2, bl2, Wr2, g2, b2,
           Wl3, bl3, Wr3, Wlin, blin):
    ei = edge_index.astype(jnp.int32)
    src = ei[0]
    dst = ei[1]
    src_r = src.reshape(NS, NGRP, G, CH)
    dst_r = dst.reshape(NS, NGRP, G, CH)
    dst_r2 = dst.reshape(2 * NS, DNCH, DCH)
    zeros_lo = jnp.zeros((N, HH), jnp.float32)
    ones40 = jnp.ones((DCH, HH), jnp.float32)
    x_lo = x[:, :HH]
    x_hi = x[:, HH:]

    bl1r, bl2r, bl3r = (v.reshape(1, H) for v in (bl1, bl2, bl3))
    g1r, b1r = g1.reshape(1, H), b1.reshape(1, H)
    g2r, b2r = g2.reshape(1, H), b2.reshape(1, H)
    blinr = blin.reshape(1, H)

    d0, d1 = _sc_degree(dst_r2, zeros_lo, ones40)
    deg = d0[:, :1] + d1[:, :1]

    a1lo, a1hi = _sc_aggregate(x_lo, x_hi, src_r, dst_r, zeros_lo)
    h1lo, h1hi = _tc_layer(a1lo, a1hi, x_lo, x_hi, deg, Wl1, bl1r, Wr1, g1r, b1r)

    a2lo, a2hi = _sc_aggregate(h1lo, h1hi, src_r, dst_r, zeros_lo)
    h2lo, h2hi = _tc_layer(a2lo, a2hi, h1lo, h1hi, deg, Wl2, bl2r, Wr2, g2r, b2r)

    a3lo, a3hi = _sc_aggregate(h2lo, h2hi, src_r, dst_r, zeros_lo)
    return _tc_final(a3lo, a3hi, h2lo, h2hi, deg, Wl3, bl3r, Wr3, Wlin, blinr)
